# baseline (speedup 1.0000x reference)
import functools

import jax
import jax.numpy as jnp
from jax import lax
from jax.experimental import pallas as pl
from jax.experimental.pallas import tpu as pltpu
from jax.experimental.pallas import tpu_sc as plsc

NC = 2
NS = 16
L = 16
NW = NC * NS
B = 16384
D = 32
BPW = B // NW
CH = 128
NCH = BPW // CH
NBLK = BPW // L


def _mf_body(uid_h, iid_h, uemb_h, iemb_h, ub_h, ib_h, out_h,
             uid_v, iid_v, urows, irows, ub_v, ib_v, out_v, sem):
    cid = lax.axis_index("c")
    sid = lax.axis_index("s")
    wid = sid * NC + cid
    base = wid * BPW

    pltpu.sync_copy(uid_h.at[pl.ds(wid * NCH, NCH)], uid_v)
    pltpu.sync_copy(iid_h.at[pl.ds(wid * NCH, NCH)], iid_v)

    cps = []
    for k in range(NCH):
        sl = pl.ds(k * CH, CH)
        cps.append(pltpu.async_copy(uemb_h.at[uid_v.at[k]], urows.at[sl], sem))
        cps.append(pltpu.async_copy(iemb_h.at[iid_v.at[k]], irows.at[sl], sem))
        cps.append(pltpu.async_copy(ub_h.at[uid_v.at[k]], ub_v.at[sl], sem))
        cps.append(pltpu.async_copy(ib_h.at[iid_v.at[k]], ib_v.at[sl], sem))
    for cp in cps:
        cp.wait()

    iota = lax.iota(jnp.int32, L)

    cols = [jnp.full((L,), c, jnp.int32) for c in range(D)]

    def block(j, carry):
        rows = j * L + iota
        accs = [ub_v[pl.ds(j * L, L)] + ib_v[pl.ds(j * L, L)],
                jnp.zeros((L,), jnp.float32),
                jnp.zeros((L,), jnp.float32),
                jnp.zeros((L,), jnp.float32)]
        for c in range(D):
            uu = plsc.load_gather(urows, [rows, cols[c]])
            ii = plsc.load_gather(irows, [rows, cols[c]])
            accs[c % 4] = accs[c % 4] + uu * ii
        acc = (accs[0] + accs[1]) + (accs[2] + accs[3])
        out_v[pl.ds(j * L, L)] = 1.0 / (1.0 + jnp.exp(-acc))
        return carry

    lax.fori_loop(0, NBLK, block, 0)
    pltpu.sync_copy(out_v, out_h.at[pl.ds(base, BPW)])


@jax.jit
def kernel(user_ids, item_ids, user_emb, item_emb, user_bias, item_bias):
    uid = user_ids.astype(jnp.int32).reshape(B // CH, CH)
    iid = item_ids.astype(jnp.int32).reshape(B // CH, CH)
    ub = user_bias.reshape(-1)
    ib = item_bias.reshape(-1)

    mesh = plsc.VectorSubcoreMesh(core_axis_name="c", subcore_axis_name="s",
                                  num_cores=NC, num_subcores=NS)
    run = pl.kernel(
        _mf_body,
        out_type=jax.ShapeDtypeStruct((B,), jnp.float32),
        mesh=mesh,
        compiler_params=pltpu.CompilerParams(needs_layout_passes=False,
                                             use_tc_tiling_on_sc=False),
        scratch_types=[
            pltpu.VMEM((NCH, CH), jnp.int32),
            pltpu.VMEM((NCH, CH), jnp.int32),
            pltpu.VMEM((BPW, D), jnp.float32),
            pltpu.VMEM((BPW, D), jnp.float32),
            pltpu.VMEM((BPW,), jnp.float32),
            pltpu.VMEM((BPW,), jnp.float32),
            pltpu.VMEM((BPW,), jnp.float32),
            pltpu.SemaphoreType.DMA,
        ],
    )
    return run(uid, iid, user_emb, item_emb, ub, ib)


# speedup vs baseline: 2.2854x; 2.2854x over previous
import functools

import jax
import jax.numpy as jnp
from jax import lax
from jax.experimental import pallas as pl
from jax.experimental.pallas import tpu as pltpu
from jax.experimental.pallas import tpu_sc as plsc

NC = 2
NS = 16
L = 16
NW = NC * NS
B = 16384
D = 32
BPW = B // NW
CH = 128
NCH = BPW // CH
NBLK = BPW // L
SUB = 4


def _bias_body(uid_h, iid_h, ub_h, ib_h, bsum_h,
               uid_v, iid_v, ub_v, ib_v, out_v, bsem):
    cid = lax.axis_index("c")
    sid = lax.axis_index("s")
    wid = sid * NC + cid
    base = wid * BPW

    pltpu.sync_copy(uid_h.at[pl.ds(base, BPW)], uid_v)
    pltpu.sync_copy(iid_h.at[pl.ds(base, BPW)], iid_v)
    bcps = []
    for k in range(NCH):
        sl = pl.ds(k * CH, CH)
        bcps.append(pltpu.async_copy(ub_h.at[uid_v.at[sl]], ub_v.at[sl], bsem))
        bcps.append(pltpu.async_copy(ib_h.at[iid_v.at[sl]], ib_v.at[sl], bsem))
    for cp in bcps:
        cp.wait()

    def blk(j, carry):
        sl = pl.ds(j * L, L)
        out_v[sl] = ub_v[sl] + ib_v[sl]
        return carry

    lax.fori_loop(0, NBLK, blk, 0)
    pltpu.sync_copy(out_v, bsum_h.at[pl.ds(base, BPW)])


def _mf_body(uid_h, iid_h, uembT_h, iembT_h, bsum_h, out_h,
             uid_v, iid_v, buf_u, buf_i, pcols, bsum_v, out_v, sem):
    cid = lax.axis_index("c")
    sid = lax.axis_index("s")
    wid = sid * NC + cid
    base = wid * BPW

    pltpu.sync_copy(uid_h.at[pl.ds(base, BPW)], uid_v)
    pltpu.sync_copy(iid_h.at[pl.ds(base, BPW)], iid_v)
    pltpu.sync_copy(bsum_h.at[pl.ds(base, BPW)], bsum_v)

    iota = lax.iota(jnp.int32, L)

    def idblock(g, carry):
        uvec = uid_v[pl.ds(g * L, L)]
        ivec = iid_v[pl.ds(g * L, L)]
        for h in range(L // SUB):
            cps = []
            for t in range(SUB):
                u = uvec[h * SUB + t]
                i = ivec[h * SUB + t]
                ucol = lax.shift_right_logical(u, 7)
                icol = lax.shift_right_logical(i, 7)
                dst = pl.ds(D * t, D)
                cps.append(pltpu.async_copy(
                    uembT_h.at[:, pl.ds(ucol * 128, 128)], buf_u.at[dst], sem))
                cps.append(pltpu.async_copy(
                    iembT_h.at[:, pl.ds(icol * 128, 128)], buf_i.at[dst], sem))
            for cp in cps:
                cp.wait()
            for t in range(SUB):
                r = g * L + h * SUB + t
                urem = uvec[h * SUB + t] & 127
                irem = ivec[h * SUB + t] & 127
                ucolv = jnp.zeros((L,), jnp.int32) + urem
                icolv = jnp.zeros((L,), jnp.int32) + irem
                u0 = plsc.load_gather(buf_u, [D * t + iota, ucolv])
                u1 = plsc.load_gather(buf_u, [D * t + L + iota, ucolv])
                i0 = plsc.load_gather(buf_i, [D * t + iota, icolv])
                i1 = plsc.load_gather(buf_i, [D * t + L + iota, icolv])
                p = u0 * i0 + u1 * i1
                plsc.store_scatter(pcols, [iota * BPW + r], p)
        return carry

    lax.fori_loop(0, NBLK, idblock, 0)

    def outblock(j, carry):
        sl = pl.ds(j * L, L)
        accs = [bsum_v[sl],
                jnp.zeros((L,), jnp.float32),
                jnp.zeros((L,), jnp.float32),
                jnp.zeros((L,), jnp.float32)]
        for lane in range(L):
            accs[lane % 4] = accs[lane % 4] + pcols[pl.ds(lane * BPW + j * L, L)]
        acc = (accs[0] + accs[1]) + (accs[2] + accs[3])
        out_v[sl] = 1.0 / (1.0 + jnp.exp(-acc))
        return carry

    lax.fori_loop(0, NBLK, outblock, 0)
    pltpu.sync_copy(out_v, out_h.at[pl.ds(base, BPW)])


@jax.jit
def kernel(user_ids, item_ids, user_emb, item_emb, user_bias, item_bias):
    uid = user_ids.astype(jnp.int32)
    iid = item_ids.astype(jnp.int32)
    ub = user_bias.reshape(-1)
    ib = item_bias.reshape(-1)

    mesh = plsc.VectorSubcoreMesh(core_axis_name="c", subcore_axis_name="s",
                                  num_cores=NC, num_subcores=NS)
    bias_run = pl.kernel(
        _bias_body,
        out_type=jax.ShapeDtypeStruct((B,), jnp.float32),
        mesh=mesh,
        compiler_params=pltpu.CompilerParams(needs_layout_passes=False,
                                             use_tc_tiling_on_sc=False),
        scratch_types=[
            pltpu.VMEM((BPW,), jnp.int32),
            pltpu.VMEM((BPW,), jnp.int32),
            pltpu.VMEM((BPW,), jnp.float32),
            pltpu.VMEM((BPW,), jnp.float32),
            pltpu.VMEM((BPW,), jnp.float32),
            pltpu.SemaphoreType.DMA,
        ],
    )
    bsum = bias_run(uid, iid, ub, ib)

    run = pl.kernel(
        _mf_body,
        out_type=jax.ShapeDtypeStruct((B,), jnp.float32),
        mesh=mesh,
        compiler_params=pltpu.CompilerParams(needs_layout_passes=False,
                                             use_tc_tiling_on_sc=True),
        scratch_types=[
            pltpu.VMEM((BPW,), jnp.int32),
            pltpu.VMEM((BPW,), jnp.int32),
            pltpu.VMEM((SUB * D, 128), jnp.float32),
            pltpu.VMEM((SUB * D, 128), jnp.float32),
            pltpu.VMEM((L * BPW,), jnp.float32),
            pltpu.VMEM((BPW,), jnp.float32),
            pltpu.VMEM((BPW,), jnp.float32),
            pltpu.SemaphoreType.DMA,
        ],
    )
    return run(uid, iid, user_emb.T, item_emb.T, bsum)


# speedup vs baseline: 2.6605x; 1.1641x over previous
import functools

import jax
import jax.numpy as jnp
from jax import lax
from jax.experimental import pallas as pl
from jax.experimental.pallas import tpu as pltpu
from jax.experimental.pallas import tpu_sc as plsc

NC = 2
NS = 16
L = 16
NW = NC * NS
B = 16384
D = 32
BPW = B // NW
CH = 128
NCH = BPW // CH
NBLK = BPW // L
SUB = 4


def _bias_body(uid_h, iid_h, ub_h, ib_h, bsum_h,
               uid_v, iid_v, ub_v, ib_v, out_v, bsem):
    cid = lax.axis_index("c")
    sid = lax.axis_index("s")
    wid = sid * NC + cid
    base = wid * BPW

    pltpu.sync_copy(uid_h.at[pl.ds(base, BPW)], uid_v)
    pltpu.sync_copy(iid_h.at[pl.ds(base, BPW)], iid_v)
    bcps = []
    for k in range(NCH):
        sl = pl.ds(k * CH, CH)
        bcps.append(pltpu.async_copy(ub_h.at[uid_v.at[sl]], ub_v.at[sl], bsem))
        bcps.append(pltpu.async_copy(ib_h.at[iid_v.at[sl]], ib_v.at[sl], bsem))
    for cp in bcps:
        cp.wait()

    def blk(j, carry):
        sl = pl.ds(j * L, L)
        out_v[sl] = ub_v[sl] + ib_v[sl]
        return carry

    lax.fori_loop(0, NBLK, blk, 0)
    pltpu.sync_copy(out_v, bsum_h.at[pl.ds(base, BPW)])


def _mf_body(uid_h, iid_h, uembT_h, iembT_h, bsum_h, out_h,
             uid_v, iid_v, buf_u, buf_i, pcols, bsum_v, out_v, sem0, sem1):
    cid = lax.axis_index("c")
    sid = lax.axis_index("s")
    wid = sid * NC + cid
    base = wid * BPW

    pltpu.sync_copy(uid_h.at[pl.ds(base, BPW)], uid_v)
    pltpu.sync_copy(iid_h.at[pl.ds(base, BPW)], iid_v)
    pltpu.sync_copy(bsum_h.at[pl.ds(base, BPW)], bsum_v)

    iota = lax.iota(jnp.int32, L)
    sems = [sem0, sem1]

    def fire(uvec, ivec, h):
        par = h % 2
        for t in range(SUB):
            u = uvec[h * SUB + t]
            i = ivec[h * SUB + t]
            ucol = lax.shift_right_logical(u, 7)
            icol = lax.shift_right_logical(i, 7)
            dst = pl.ds((par * SUB + t) * D, D)
            pltpu.async_copy(uembT_h.at[:, pl.ds(ucol * 128, 128)],
                             buf_u.at[dst], sems[par])
            pltpu.async_copy(iembT_h.at[:, pl.ds(icol * 128, 128)],
                             buf_i.at[dst], sems[par])

    def drain(h):
        par = h % 2
        for t in range(SUB):
            dst = pl.ds((par * SUB + t) * D, D)
            pltpu.make_async_copy(uembT_h.at[:, pl.ds(0, 128)],
                                  buf_u.at[dst], sems[par]).wait()
            pltpu.make_async_copy(iembT_h.at[:, pl.ds(0, 128)],
                                  buf_i.at[dst], sems[par]).wait()

    def extract(uvec, ivec, g, h):
        par = h % 2
        for t in range(SUB):
            r = g * L + h * SUB + t
            urem = uvec[h * SUB + t] & 127
            irem = ivec[h * SUB + t] & 127
            ucolv = jnp.zeros((L,), jnp.int32) + urem
            icolv = jnp.zeros((L,), jnp.int32) + irem
            row0 = (par * SUB + t) * D
            u0 = plsc.load_gather(buf_u, [row0 + iota, ucolv])
            u1 = plsc.load_gather(buf_u, [row0 + L + iota, ucolv])
            i0 = plsc.load_gather(buf_i, [row0 + iota, icolv])
            i1 = plsc.load_gather(buf_i, [row0 + L + iota, icolv])
            p = u0 * i0 + u1 * i1
            plsc.store_scatter(pcols, [iota * BPW + r], p)

    uvec0 = uid_v[pl.ds(0, L)]
    ivec0 = iid_v[pl.ds(0, L)]
    fire(uvec0, ivec0, 0)

    def idblock(g, carry):
        uvec = uid_v[pl.ds(g * L, L)]
        ivec = iid_v[pl.ds(g * L, L)]
        nxt_sl = pl.ds(jnp.minimum(g + 1, NBLK - 1) * L, L)
        uvec_n = uid_v[nxt_sl]
        ivec_n = iid_v[nxt_sl]
        for h in range(L // SUB):
            if h + 1 < L // SUB:
                fire(uvec, ivec, h + 1)
            else:
                fire(uvec_n, ivec_n, 0)
            drain(h)
            extract(uvec, ivec, g, h)
        return carry

    lax.fori_loop(0, NBLK, idblock, 0)
    drain(0)

    def outblock(j, carry):
        sl = pl.ds(j * L, L)
        accs = [bsum_v[sl],
                jnp.zeros((L,), jnp.float32),
                jnp.zeros((L,), jnp.float32),
                jnp.zeros((L,), jnp.float32)]
        for lane in range(L):
            accs[lane % 4] = accs[lane % 4] + pcols[pl.ds(lane * BPW + j * L, L)]
        acc = (accs[0] + accs[1]) + (accs[2] + accs[3])
        out_v[sl] = 1.0 / (1.0 + jnp.exp(-acc))
        return carry

    lax.fori_loop(0, NBLK, outblock, 0)
    pltpu.sync_copy(out_v, out_h.at[pl.ds(base, BPW)])


@jax.jit
def kernel(user_ids, item_ids, user_emb, item_emb, user_bias, item_bias):
    uid = user_ids.astype(jnp.int32)
    iid = item_ids.astype(jnp.int32)
    ub = user_bias.reshape(-1)
    ib = item_bias.reshape(-1)

    mesh = plsc.VectorSubcoreMesh(core_axis_name="c", subcore_axis_name="s",
                                  num_cores=NC, num_subcores=NS)
    bias_run = pl.kernel(
        _bias_body,
        out_type=jax.ShapeDtypeStruct((B,), jnp.float32),
        mesh=mesh,
        compiler_params=pltpu.CompilerParams(needs_layout_passes=False,
                                             use_tc_tiling_on_sc=False),
        scratch_types=[
            pltpu.VMEM((BPW,), jnp.int32),
            pltpu.VMEM((BPW,), jnp.int32),
            pltpu.VMEM((BPW,), jnp.float32),
            pltpu.VMEM((BPW,), jnp.float32),
            pltpu.VMEM((BPW,), jnp.float32),
            pltpu.SemaphoreType.DMA,
        ],
    )
    bsum = bias_run(uid, iid, ub, ib)

    run = pl.kernel(
        _mf_body,
        out_type=jax.ShapeDtypeStruct((B,), jnp.float32),
        mesh=mesh,
        compiler_params=pltpu.CompilerParams(needs_layout_passes=False,
                                             use_tc_tiling_on_sc=True),
        scratch_types=[
            pltpu.VMEM((BPW,), jnp.int32),
            pltpu.VMEM((BPW,), jnp.int32),
            pltpu.VMEM((2 * SUB * D, 128), jnp.float32),
            pltpu.VMEM((2 * SUB * D, 128), jnp.float32),
            pltpu.VMEM((L * BPW,), jnp.float32),
            pltpu.VMEM((BPW,), jnp.float32),
            pltpu.VMEM((BPW,), jnp.float32),
            pltpu.SemaphoreType.DMA,
            pltpu.SemaphoreType.DMA,
        ],
    )
    return run(uid, iid, user_emb.T, item_emb.T, bsum)


# speedup vs baseline: 2.7322x; 1.0269x over previous
import functools

import jax
import jax.numpy as jnp
from jax import lax
from jax.experimental import pallas as pl
from jax.experimental.pallas import tpu as pltpu
from jax.experimental.pallas import tpu_sc as plsc

NC = 2
NS = 16
L = 16
NW = NC * NS
B = 16384
D = 32
BPW = B // NW
NBLK = BPW // L
SUB = 4


def _mf_body(uid_h, iid_h, uembT_h, iembT_h, ub_h, ib_h, out_h,
             uid_v, iid_v, buf_u, buf_i, pcols, bb_u, bb_i, out_v,
             sem0, sem1, bsem):
    cid = lax.axis_index("c")
    sid = lax.axis_index("s")
    wid = sid * NC + cid
    base = wid * BPW

    pltpu.sync_copy(uid_h.at[pl.ds(base, BPW)], uid_v)
    pltpu.sync_copy(iid_h.at[pl.ds(base, BPW)], iid_v)

    iota = lax.iota(jnp.int32, L)
    sems = [sem0, sem1]

    def fire(uvec, ivec, h):
        par = h % 2
        for t in range(SUB):
            u = uvec[h * SUB + t]
            i = ivec[h * SUB + t]
            ucol = lax.shift_right_logical(u, 7)
            icol = lax.shift_right_logical(i, 7)
            dst = pl.ds((par * SUB + t) * D, D)
            pltpu.async_copy(uembT_h.at[:, pl.ds(ucol * 128, 128)],
                             buf_u.at[dst], sems[par])
            pltpu.async_copy(iembT_h.at[:, pl.ds(icol * 128, 128)],
                             buf_i.at[dst], sems[par])

    def fire_bias(uvec, ivec, g, h):
        for t in range(SUB):
            r = g * L + h * SUB + t
            u8 = pl.multiple_of(uvec[h * SUB + t] & ~7, 8)
            i8 = pl.multiple_of(ivec[h * SUB + t] & ~7, 8)
            pltpu.async_copy(ub_h.at[pl.ds(u8, 8)], bb_u.at[pl.ds(r * 8, 8)],
                             bsem)
            pltpu.async_copy(ib_h.at[pl.ds(i8, 8)], bb_i.at[pl.ds(r * 8, 8)],
                             bsem)

    def drain(h):
        par = h % 2
        for t in range(SUB):
            dst = pl.ds((par * SUB + t) * D, D)
            pltpu.make_async_copy(uembT_h.at[:, pl.ds(0, 128)],
                                  buf_u.at[dst], sems[par]).wait()
            pltpu.make_async_copy(iembT_h.at[:, pl.ds(0, 128)],
                                  buf_i.at[dst], sems[par]).wait()

    def extract(uvec, ivec, g, h):
        par = h % 2
        for t in range(SUB):
            r = g * L + h * SUB + t
            urem = uvec[h * SUB + t] & 127
            irem = ivec[h * SUB + t] & 127
            ucolv = jnp.zeros((L,), jnp.int32) + urem
            icolv = jnp.zeros((L,), jnp.int32) + irem
            row0 = (par * SUB + t) * D
            u0 = plsc.load_gather(buf_u, [row0 + iota, ucolv])
            u1 = plsc.load_gather(buf_u, [row0 + L + iota, ucolv])
            i0 = plsc.load_gather(buf_i, [row0 + iota, icolv])
            i1 = plsc.load_gather(buf_i, [row0 + L + iota, icolv])
            p = u0 * i0 + u1 * i1
            plsc.store_scatter(pcols, [iota * BPW + r], p)

    uvec0 = uid_v[pl.ds(0, L)]
    ivec0 = iid_v[pl.ds(0, L)]
    fire(uvec0, ivec0, 0)

    def idblock(g, carry):
        uvec = uid_v[pl.ds(g * L, L)]
        ivec = iid_v[pl.ds(g * L, L)]
        nxt_sl = pl.ds(jnp.minimum(g + 1, NBLK - 1) * L, L)
        uvec_n = uid_v[nxt_sl]
        ivec_n = iid_v[nxt_sl]
        for h in range(L // SUB):
            if h + 1 < L // SUB:
                fire(uvec, ivec, h + 1)
            else:
                fire(uvec_n, ivec_n, 0)
            fire_bias(uvec, ivec, g, h)
            drain(h)
            extract(uvec, ivec, g, h)
        return carry

    lax.fori_loop(0, NBLK, idblock, 0)
    drain(0)
    pltpu.make_async_copy(ub_h.at[pl.ds(0, BPW * 8)], bb_u, bsem).wait()
    pltpu.make_async_copy(ib_h.at[pl.ds(0, BPW * 8)], bb_i, bsem).wait()

    def outblock(j, carry):
        sl = pl.ds(j * L, L)
        uvec = uid_v[sl]
        ivec = iid_v[sl]
        ub16 = plsc.load_gather(bb_u, [(j * L + iota) * 8 + (uvec & 7)])
        ib16 = plsc.load_gather(bb_i, [(j * L + iota) * 8 + (ivec & 7)])
        accs = [ub16 + ib16,
                jnp.zeros((L,), jnp.float32),
                jnp.zeros((L,), jnp.float32),
                jnp.zeros((L,), jnp.float32)]
        for lane in range(L):
            accs[lane % 4] = accs[lane % 4] + pcols[pl.ds(lane * BPW + j * L, L)]
        acc = (accs[0] + accs[1]) + (accs[2] + accs[3])
        out_v[sl] = 1.0 / (1.0 + jnp.exp(-acc))
        return carry

    lax.fori_loop(0, NBLK, outblock, 0)
    pltpu.sync_copy(out_v, out_h.at[pl.ds(base, BPW)])


@jax.jit
def kernel(user_ids, item_ids, user_emb, item_emb, user_bias, item_bias):
    uid = user_ids.astype(jnp.int32)
    iid = item_ids.astype(jnp.int32)
    ub = user_bias.reshape(-1)
    ib = item_bias.reshape(-1)

    mesh = plsc.VectorSubcoreMesh(core_axis_name="c", subcore_axis_name="s",
                                  num_cores=NC, num_subcores=NS)
    run = pl.kernel(
        _mf_body,
        out_type=jax.ShapeDtypeStruct((B,), jnp.float32),
        mesh=mesh,
        compiler_params=pltpu.CompilerParams(needs_layout_passes=False,
                                             use_tc_tiling_on_sc=True),
        scratch_types=[
            pltpu.VMEM((BPW,), jnp.int32),
            pltpu.VMEM((BPW,), jnp.int32),
            pltpu.VMEM((2 * SUB * D, 128), jnp.float32),
            pltpu.VMEM((2 * SUB * D, 128), jnp.float32),
            pltpu.VMEM((L * BPW,), jnp.float32),
            pltpu.VMEM((BPW * 8,), jnp.float32),
            pltpu.VMEM((BPW * 8,), jnp.float32),
            pltpu.VMEM((BPW,), jnp.float32),
            pltpu.SemaphoreType.DMA,
            pltpu.SemaphoreType.DMA,
            pltpu.SemaphoreType.DMA,
        ],
    )
    return run(uid, iid, user_emb.T, item_emb.T, ub, ib)


# speedup vs baseline: 2.9248x; 1.0705x over previous
import functools

import jax
import jax.numpy as jnp
from jax import lax
from jax.experimental import pallas as pl
from jax.experimental.pallas import tpu as pltpu
from jax.experimental.pallas import tpu_sc as plsc

NC = 2
NS = 16
L = 16
NW = NC * NS
B = 16384
D = 32
BPW = B // NW
NBLK = BPW // L
SUB = 2
NSETS = 4
AHEAD = 3


def _mf_body(uid_h, iid_h, uembT_h, iembT_h, ub_h, ib_h, out_h,
             uid_v, iid_v, buf_u, buf_i, pcols, bb_u, bb_i, out_v,
             sem0, sem1, sem2, sem3, bsem):
    cid = lax.axis_index("c")
    sid = lax.axis_index("s")
    wid = sid * NC + cid
    base = wid * BPW

    pltpu.sync_copy(uid_h.at[pl.ds(base, BPW)], uid_v)
    pltpu.sync_copy(iid_h.at[pl.ds(base, BPW)], iid_v)

    iota = lax.iota(jnp.int32, L)
    sems = [sem0, sem1, sem2, sem3]

    def fire(uvec, ivec, h):
        par = h % NSETS
        for t in range(SUB):
            u = uvec[h * SUB + t]
            i = ivec[h * SUB + t]
            ucol = lax.shift_right_logical(u, 7)
            icol = lax.shift_right_logical(i, 7)
            dst = pl.ds((par * SUB + t) * D, D)
            pltpu.async_copy(uembT_h.at[:, pl.ds(ucol * 128, 128)],
                             buf_u.at[dst], sems[par])
            pltpu.async_copy(iembT_h.at[:, pl.ds(icol * 128, 128)],
                             buf_i.at[dst], sems[par])

    def fire_bias(uvec, ivec, g, h):
        for t in range(SUB):
            r = g * L + h * SUB + t
            u8 = pl.multiple_of(uvec[h * SUB + t] & ~7, 8)
            i8 = pl.multiple_of(ivec[h * SUB + t] & ~7, 8)
            pltpu.async_copy(ub_h.at[pl.ds(u8, 8)], bb_u.at[pl.ds(r * 8, 8)],
                             bsem)
            pltpu.async_copy(ib_h.at[pl.ds(i8, 8)], bb_i.at[pl.ds(r * 8, 8)],
                             bsem)

    def drain(h):
        par = h % NSETS
        for t in range(SUB):
            dst = pl.ds((par * SUB + t) * D, D)
            pltpu.make_async_copy(uembT_h.at[:, pl.ds(0, 128)],
                                  buf_u.at[dst], sems[par]).wait()
            pltpu.make_async_copy(iembT_h.at[:, pl.ds(0, 128)],
                                  buf_i.at[dst], sems[par]).wait()

    def extract(uvec, ivec, g, h):
        par = h % NSETS
        for t in range(SUB):
            r = g * L + h * SUB + t
            urem = uvec[h * SUB + t] & 127
            irem = ivec[h * SUB + t] & 127
            ucolv = jnp.zeros((L,), jnp.int32) + urem
            icolv = jnp.zeros((L,), jnp.int32) + irem
            row0 = (par * SUB + t) * D
            u0 = plsc.load_gather(buf_u, [row0 + iota, ucolv])
            u1 = plsc.load_gather(buf_u, [row0 + L + iota, ucolv])
            i0 = plsc.load_gather(buf_i, [row0 + iota, icolv])
            i1 = plsc.load_gather(buf_i, [row0 + L + iota, icolv])
            p = u0 * i0 + u1 * i1
            plsc.store_scatter(pcols, [iota * BPW + r], p)

    uvec0 = uid_v[pl.ds(0, L)]
    ivec0 = iid_v[pl.ds(0, L)]
    for s in range(AHEAD):
        fire(uvec0, ivec0, s)

    def idblock(g, carry):
        uvec = uid_v[pl.ds(g * L, L)]
        ivec = iid_v[pl.ds(g * L, L)]
        nxt_sl = pl.ds(jnp.minimum(g + 1, NBLK - 1) * L, L)
        uvec_n = uid_v[nxt_sl]
        ivec_n = iid_v[nxt_sl]
        nsub = L // SUB
        for h in range(nsub):
            if h + AHEAD < nsub:
                fire(uvec, ivec, h + AHEAD)
            else:
                fire(uvec_n, ivec_n, h + AHEAD - nsub)
            fire_bias(uvec, ivec, g, h)
            drain(h)
            extract(uvec, ivec, g, h)
        return carry

    lax.fori_loop(0, NBLK, idblock, 0)
    for s in range(AHEAD):
        drain(s)
    pltpu.make_async_copy(ub_h.at[pl.ds(0, BPW * 8)], bb_u, bsem).wait()
    pltpu.make_async_copy(ib_h.at[pl.ds(0, BPW * 8)], bb_i, bsem).wait()

    def outblock(j, carry):
        sl = pl.ds(j * L, L)
        uvec = uid_v[sl]
        ivec = iid_v[sl]
        ub16 = plsc.load_gather(bb_u, [(j * L + iota) * 8 + (uvec & 7)])
        ib16 = plsc.load_gather(bb_i, [(j * L + iota) * 8 + (ivec & 7)])
        accs = [ub16 + ib16,
                jnp.zeros((L,), jnp.float32),
                jnp.zeros((L,), jnp.float32),
                jnp.zeros((L,), jnp.float32)]
        for lane in range(L):
            accs[lane % 4] = accs[lane % 4] + pcols[pl.ds(lane * BPW + j * L, L)]
        acc = (accs[0] + accs[1]) + (accs[2] + accs[3])
        out_v[sl] = 1.0 / (1.0 + jnp.exp(-acc))
        return carry

    lax.fori_loop(0, NBLK, outblock, 0)
    pltpu.sync_copy(out_v, out_h.at[pl.ds(base, BPW)])


@jax.jit
def kernel(user_ids, item_ids, user_emb, item_emb, user_bias, item_bias):
    uid = user_ids.astype(jnp.int32)
    iid = item_ids.astype(jnp.int32)
    ub = user_bias.reshape(-1)
    ib = item_bias.reshape(-1)

    mesh = plsc.VectorSubcoreMesh(core_axis_name="c", subcore_axis_name="s",
                                  num_cores=NC, num_subcores=NS)
    run = pl.kernel(
        _mf_body,
        out_type=jax.ShapeDtypeStruct((B,), jnp.float32),
        mesh=mesh,
        compiler_params=pltpu.CompilerParams(needs_layout_passes=False,
                                             use_tc_tiling_on_sc=True),
        scratch_types=[
            pltpu.VMEM((BPW,), jnp.int32),
            pltpu.VMEM((BPW,), jnp.int32),
            pltpu.VMEM((NSETS * SUB * D, 128), jnp.float32),
            pltpu.VMEM((NSETS * SUB * D, 128), jnp.float32),
            pltpu.VMEM((L * BPW,), jnp.float32),
            pltpu.VMEM((BPW * 8,), jnp.float32),
            pltpu.VMEM((BPW * 8,), jnp.float32),
            pltpu.VMEM((BPW,), jnp.float32),
            pltpu.SemaphoreType.DMA,
            pltpu.SemaphoreType.DMA,
            pltpu.SemaphoreType.DMA,
            pltpu.SemaphoreType.DMA,
            pltpu.SemaphoreType.DMA,
        ],
    )
    return run(uid, iid, user_emb.T, item_emb.T, ub, ib)


# speedup vs baseline: 2.9274x; 1.0009x over previous
import functools

import jax
import jax.numpy as jnp
from jax import lax
from jax.experimental import pallas as pl
from jax.experimental.pallas import tpu as pltpu
from jax.experimental.pallas import tpu_sc as plsc

NC = 2
NS = 16
L = 16
NW = NC * NS
B = 16384
D = 32
BPW = B // NW
NBLK = BPW // L
SUB = 2
NSETS = 4
AHEAD = 3


def _mf_body(uid_h, iid_h, uembT_h, iembT_h, ub_h, ib_h, out_h,
             uid_v, iid_v, buf_u, buf_i, pcols, bb_u, bb_i, out_v,
             sem0, sem1, sem2, sem3, bsem):
    cid = lax.axis_index("c")
    sid = lax.axis_index("s")
    wid = sid * NC + cid
    base = wid * BPW

    pltpu.sync_copy(uid_h.at[pl.ds(base, BPW)], uid_v)
    pltpu.sync_copy(iid_h.at[pl.ds(base, BPW)], iid_v)

    iota = lax.iota(jnp.int32, L)
    sems = [sem0, sem1, sem2, sem3]

    def fire(uvec, ivec, h):
        par = h % NSETS
        for t in range(SUB):
            u = uvec[h * SUB + t]
            i = ivec[h * SUB + t]
            ucol = lax.shift_right_logical(u, 7)
            icol = lax.shift_right_logical(i, 7)
            row0 = (par * SUB + t) * D
            for tc in range(D // 8):
                dst = pl.ds(row0 + 8 * tc, 8)
                src_r = pl.ds(8 * tc, 8)
                pltpu.async_copy(uembT_h.at[src_r, pl.ds(ucol * 128, 128)],
                                 buf_u.at[dst], sems[par])
                pltpu.async_copy(iembT_h.at[src_r, pl.ds(icol * 128, 128)],
                                 buf_i.at[dst], sems[par])

    def fire_bias(uvec, ivec, g, h):
        for t in range(SUB):
            r = g * L + h * SUB + t
            u8 = pl.multiple_of(uvec[h * SUB + t] & ~7, 8)
            i8 = pl.multiple_of(ivec[h * SUB + t] & ~7, 8)
            pltpu.async_copy(ub_h.at[pl.ds(u8, 8)], bb_u.at[pl.ds(r * 8, 8)],
                             bsem)
            pltpu.async_copy(ib_h.at[pl.ds(i8, 8)], bb_i.at[pl.ds(r * 8, 8)],
                             bsem)

    def drain(h):
        par = h % NSETS
        for t in range(SUB):
            dst = pl.ds((par * SUB + t) * D, D)
            pltpu.make_async_copy(uembT_h.at[:, pl.ds(0, 128)],
                                  buf_u.at[dst], sems[par]).wait()
            pltpu.make_async_copy(iembT_h.at[:, pl.ds(0, 128)],
                                  buf_i.at[dst], sems[par]).wait()

    def extract(uvec, ivec, g, h):
        par = h % NSETS
        for t in range(SUB):
            r = g * L + h * SUB + t
            urem = uvec[h * SUB + t] & 127
            irem = ivec[h * SUB + t] & 127
            ucolv = jnp.zeros((L,), jnp.int32) + urem
            icolv = jnp.zeros((L,), jnp.int32) + irem
            row0 = (par * SUB + t) * D
            u0 = plsc.load_gather(buf_u, [row0 + iota, ucolv])
            u1 = plsc.load_gather(buf_u, [row0 + L + iota, ucolv])
            i0 = plsc.load_gather(buf_i, [row0 + iota, icolv])
            i1 = plsc.load_gather(buf_i, [row0 + L + iota, icolv])
            p = u0 * i0 + u1 * i1
            plsc.store_scatter(pcols, [iota * BPW + r], p)

    uvec0 = uid_v[pl.ds(0, L)]
    ivec0 = iid_v[pl.ds(0, L)]
    for s in range(AHEAD):
        fire(uvec0, ivec0, s)

    def idblock(g, carry):
        uvec = uid_v[pl.ds(g * L, L)]
        ivec = iid_v[pl.ds(g * L, L)]
        nxt_sl = pl.ds(jnp.minimum(g + 1, NBLK - 1) * L, L)
        uvec_n = uid_v[nxt_sl]
        ivec_n = iid_v[nxt_sl]
        nsub = L // SUB
        for h in range(nsub):
            if h + AHEAD < nsub:
                fire(uvec, ivec, h + AHEAD)
            else:
                fire(uvec_n, ivec_n, h + AHEAD - nsub)
            fire_bias(uvec, ivec, g, h)
            drain(h)
            extract(uvec, ivec, g, h)
        return carry

    lax.fori_loop(0, NBLK, idblock, 0)
    for s in range(AHEAD):
        drain(s)
    pltpu.make_async_copy(ub_h.at[pl.ds(0, BPW * 8)], bb_u, bsem).wait()
    pltpu.make_async_copy(ib_h.at[pl.ds(0, BPW * 8)], bb_i, bsem).wait()

    def outblock(j, carry):
        sl = pl.ds(j * L, L)
        uvec = uid_v[sl]
        ivec = iid_v[sl]
        ub16 = plsc.load_gather(bb_u, [(j * L + iota) * 8 + (uvec & 7)])
        ib16 = plsc.load_gather(bb_i, [(j * L + iota) * 8 + (ivec & 7)])
        accs = [ub16 + ib16,
                jnp.zeros((L,), jnp.float32),
                jnp.zeros((L,), jnp.float32),
                jnp.zeros((L,), jnp.float32)]
        for lane in range(L):
            accs[lane % 4] = accs[lane % 4] + pcols[pl.ds(lane * BPW + j * L, L)]
        acc = (accs[0] + accs[1]) + (accs[2] + accs[3])
        out_v[sl] = 1.0 / (1.0 + jnp.exp(-acc))
        return carry

    lax.fori_loop(0, NBLK, outblock, 0)
    pltpu.sync_copy(out_v, out_h.at[pl.ds(base, BPW)])


@jax.jit
def kernel(user_ids, item_ids, user_emb, item_emb, user_bias, item_bias):
    uid = user_ids.astype(jnp.int32)
    iid = item_ids.astype(jnp.int32)
    ub = user_bias.reshape(-1)
    ib = item_bias.reshape(-1)

    mesh = plsc.VectorSubcoreMesh(core_axis_name="c", subcore_axis_name="s",
                                  num_cores=NC, num_subcores=NS)
    run = pl.kernel(
        _mf_body,
        out_type=jax.ShapeDtypeStruct((B,), jnp.float32),
        mesh=mesh,
        compiler_params=pltpu.CompilerParams(needs_layout_passes=False,
                                             use_tc_tiling_on_sc=True),
        scratch_types=[
            pltpu.VMEM((BPW,), jnp.int32),
            pltpu.VMEM((BPW,), jnp.int32),
            pltpu.VMEM((NSETS * SUB * D, 128), jnp.float32),
            pltpu.VMEM((NSETS * SUB * D, 128), jnp.float32),
            pltpu.VMEM((L * BPW,), jnp.float32),
            pltpu.VMEM((BPW * 8,), jnp.float32),
            pltpu.VMEM((BPW * 8,), jnp.float32),
            pltpu.VMEM((BPW,), jnp.float32),
            pltpu.SemaphoreType.DMA,
            pltpu.SemaphoreType.DMA,
            pltpu.SemaphoreType.DMA,
            pltpu.SemaphoreType.DMA,
            pltpu.SemaphoreType.DMA,
        ],
    )
    return run(uid, iid, user_emb.T, item_emb.T, ub, ib)
